# baseline (speedup 1.0000x reference)
import functools

import jax
import jax.numpy as jnp
from jax import lax
from jax.experimental import pallas as pl
from jax.experimental.pallas import tpu as pltpu
from jax.experimental.pallas import tpu_sc as plsc

NUM_EMBEDDINGS = 1000000
EMBEDDING_DIM = 64
TP_WORLD_SIZE = 2
NUM_EMB_PER_PART = NUM_EMBEDDINGS // TP_WORLD_SIZE
VOCAB_START = 0
VOCAB_END = NUM_EMB_PER_PART
NUM_TOKENS = 16384

NC = 2
NS = 16
NW = NC * NS
BPW = NUM_TOKENS // NW
CH = 32
NCHUNK = BPW // CH
NPACK = NUM_EMB_PER_PART // 2
PD = 2 * EMBEDDING_DIM
NBUF = 8

_mesh = plsc.VectorSubcoreMesh(core_axis_name="c", subcore_axis_name="s")


@functools.partial(
    pl.kernel,
    mesh=_mesh,
    out_type=jax.ShapeDtypeStruct((NUM_TOKENS, EMBEDDING_DIM), jnp.float32),
    scratch_types=[
        pltpu.VMEM((BPW,), jnp.int32),
        pltpu.VMEM((NCHUNK, CH), jnp.int32),
        pltpu.VMEM((BPW,), jnp.int32),
        pltpu.VMEM((BPW,), jnp.float32),
        pltpu.VMEM((NBUF * CH, PD), jnp.float32),
        pltpu.VMEM((BPW, EMBEDDING_DIM), jnp.float32),
        [pltpu.SemaphoreType.DMA] * NBUF,
    ],
    compiler_params=pltpu.CompilerParams(needs_layout_passes=False),
)
def _emb_kernel(x_hbm, w_hbm, out_hbm, idx_v, p_v, h_v, fm_v, tiles_v, rows_v, sems):
    wid = lax.axis_index("s") * NC + lax.axis_index("c")
    base = wid * BPW

    pltpu.sync_copy(x_hbm.at[pl.ds(base, BPW)], idx_v)

    ones_f = jnp.full((16,), 1.0, jnp.float32)
    zeros_f = jnp.full((16,), 0.0, jnp.float32)
    zeros_i = jnp.full((16,), 0, jnp.int32)
    span = jnp.full((16,), VOCAB_END - VOCAB_START, jnp.uint32)

    def mask_body(g, _):
        iv = idx_v[pl.ds(g * 16, 16)]
        rel = iv - VOCAB_START
        m = plsc.bitcast(rel, jnp.uint32) < span
        clamped = jnp.where(m, rel, zeros_i)
        p_v[g // (CH // 16), pl.ds((g % (CH // 16)) * 16, 16)] = clamped >> 1
        h_v[pl.ds(g * 16, 16)] = (clamped & 1) * EMBEDDING_DIM
        fm_v[pl.ds(g * 16, 16)] = jnp.where(m, ones_f, zeros_f)
        return 0

    for g in range(BPW // 16):
        mask_body(g, 0)

    def fire(k):
        return pltpu.async_copy(
            w_hbm.at[p_v.at[k]],
            tiles_v.at[pl.ds((k % NBUF) * CH, CH)],
            sems[k % NBUF],
        )

    copies = [fire(k) for k in range(NBUF)]
    for k in range(NCHUNK):
        copies[k].wait()
        s0 = (k % NBUF) * CH
        for g in range(CH // 16):
            t0 = k * CH + g * 16
            hvec = h_v[pl.ds(t0, 16)]
            fmvec = fm_v[pl.ds(t0, 16)]
            for l in range(16):
                h_l = hvec[l]
                bc = jnp.full((16,), fmvec[l], jnp.float32)
                t = t0 + l
                for j in range(EMBEDDING_DIM // 16):
                    seg = tiles_v[s0 + g * 16 + l, pl.ds(h_l + 16 * j, 16)]
                    rows_v[t, pl.ds(16 * j, 16)] = seg * bc
        if k + NBUF < NCHUNK:
            copies.append(fire(k + NBUF))

    pltpu.sync_copy(rows_v, out_hbm.at[pl.ds(base, BPW)])


def kernel(x, weight):
    w2 = weight.reshape(NPACK, PD)
    return _emb_kernel(x.astype(jnp.int32), w2)


# speedup vs baseline: 1.8169x; 1.8169x over previous
import functools

import jax
import jax.numpy as jnp
from jax import lax
from jax.experimental import pallas as pl
from jax.experimental.pallas import tpu as pltpu
from jax.experimental.pallas import tpu_sc as plsc

NUM_EMBEDDINGS = 1000000
EMBEDDING_DIM = 64
TP_WORLD_SIZE = 2
NUM_EMB_PER_PART = NUM_EMBEDDINGS // TP_WORLD_SIZE
VOCAB_START = 0
VOCAB_END = NUM_EMB_PER_PART
NUM_TOKENS = 16384

NC = 2
NS = 16
NW = NC * NS
BPW = NUM_TOKENS // NW
CH = 64
NCH = BPW // CH
DUMP = BPW

_mesh = plsc.VectorSubcoreMesh(core_axis_name="c", subcore_axis_name="s")


@functools.partial(
    pl.kernel,
    mesh=_mesh,
    out_type=jax.ShapeDtypeStruct((NUM_TOKENS, EMBEDDING_DIM), jnp.float32),
    scratch_types=[
        pltpu.VMEM((BPW,), jnp.int32),
        pltpu.VMEM((BPW,), jnp.int32),
        pltpu.VMEM((BPW,), jnp.int32),
        pltpu.VMEM((BPW, EMBEDDING_DIM), jnp.float32),
        pltpu.VMEM((BPW + 16, EMBEDDING_DIM), jnp.float32),
        [pltpu.SemaphoreType.DMA] * NCH,
    ],
    compiler_params=pltpu.CompilerParams(
        use_tc_tiling_on_sc=False, needs_layout_passes=False
    ),
)
def _emb_kernel(x_hbm, w_hbm, out_hbm, idx_v, vidx_v, vpos_v, gath_v, rows_v, sems):
    wid = lax.axis_index("s") * NC + lax.axis_index("c")
    base = wid * BPW

    pltpu.sync_copy(x_hbm.at[pl.ds(base, BPW)], idx_v)

    ones_i = jnp.full((16,), 1, jnp.int32)
    zeros_i = jnp.full((16,), 0, jnp.int32)
    dump_i = jnp.full((16,), DUMP, jnp.int32)
    span = jnp.full((16,), VOCAB_END - VOCAB_START, jnp.uint32)
    lanes = lax.iota(jnp.int32, 16)

    def prefill(g, _):
        vidx_v[pl.ds(g * 16, 16)] = zeros_i
        vpos_v[pl.ds(g * 16, 16)] = dump_i
        return 0

    lax.fori_loop(0, BPW // 16, prefill, 0)

    cnt_vec = zeros_i
    for g in range(BPW // 16):
        iv = idx_v[pl.ds(g * 16, 16)]
        rel = iv - VOCAB_START
        m = plsc.bitcast(rel, jnp.uint32) < span
        cm = plsc.cumsum(jnp.where(m, ones_i, zeros_i))
        pos = cm - 1 + cnt_vec
        plsc.store_scatter(vidx_v, [pos], rel, mask=m)
        plsc.store_scatter(vpos_v, [pos], lanes + g * 16, mask=m)
        cnt_vec = cnt_vec + plsc.all_reduce_population_count(m)
    nv = cnt_vec[0]

    copies = []
    for k in range(NCH):
        @pl.when(nv > k * CH)
        def _():
            pltpu.async_copy(
                w_hbm.at[vidx_v.at[pl.ds(k * CH, CH)]],
                gath_v.at[pl.ds(k * CH, CH)],
                sems[k],
            )
        copies.append(
            pltpu.make_async_copy(
                w_hbm.at[pl.ds(0, CH)], gath_v.at[pl.ds(k * CH, CH)], sems[k]
            )
        )

    def zero_body(g, _):
        for j in range(EMBEDDING_DIM // 16):
            rows_v[g, pl.ds(j * 16, 16)] = jnp.full((16,), 0.0, jnp.float32)
        return 0

    lax.fori_loop(0, BPW + 16, zero_body, 0)

    for k in range(NCH):
        @pl.when(nv > k * CH)
        def _():
            copies[k].wait()
            for g in range(CH // 16):
                pvec = vpos_v[pl.ds(k * CH + g * 16, 16)]
                for l in range(16):
                    p = pvec[l]
                    src = k * CH + g * 16 + l
                    for j in range(EMBEDDING_DIM // 16):
                        rows_v[p, pl.ds(j * 16, 16)] = gath_v[
                            src, pl.ds(j * 16, 16)
                        ]

    pltpu.sync_copy(rows_v.at[pl.ds(0, BPW)], out_hbm.at[pl.ds(base, BPW)])


def kernel(x, weight):
    return _emb_kernel(x.astype(jnp.int32), weight)
